# Initial kernel scaffold; baseline (speedup 1.0000x reference)
import jax
import jax.numpy as jnp
from jax.experimental import pallas as pl


def kernel(x):
    raise NotImplementedError("write your pallas kernel here")


# speedup vs baseline: 19.7333x; 19.7333x over previous
import jax
import jax.numpy as jnp
from jax.experimental import pallas as pl
from jax.experimental.pallas import tpu as pltpu

_B, _C, _H, _W = 4, 384, 224, 224
_S = 16
_T = 56
_G = 32
_CG = _C // _G


def _shift_kernel(x_ref, o_ref):
    g = pl.program_id(1)
    s = jnp.where(g >= _S, g - _S, 0)
    a = s // 4
    r = s % 4
    v = x_ref[0, 0]
    v1 = pltpu.roll(v, (_W - _T * r) % _W, axis=2)
    ha = (_H - _T * a) % _H
    hb = (_H - _T * (a + 1)) % _H
    va = pltpu.roll(v1, ha, axis=1)
    vb = pltpu.roll(v1, hb, axis=1)
    lane = jax.lax.broadcasted_iota(jnp.int32, v.shape, 2)
    o_ref[0, 0] = jnp.where(lane >= _W - _T * r, vb, va)


def kernel(x):
    xr = x.reshape(_B, _G, _CG, _H, _W)
    spec = pl.BlockSpec(
        (1, 1, _CG, _H, _W), lambda b, g: (b, g, 0, 0, 0)
    )
    out = pl.pallas_call(
        _shift_kernel,
        grid=(_B, _G),
        in_specs=[spec],
        out_specs=spec,
        out_shape=jax.ShapeDtypeStruct((_B, _G, _CG, _H, _W), x.dtype),
        compiler_params=pltpu.CompilerParams(
            dimension_semantics=("arbitrary", "arbitrary"),
        ),
    )(xr)
    return out.reshape(_B, _C, _H, _W)


# speedup vs baseline: 24.2999x; 1.2314x over previous
import jax
import jax.numpy as jnp
from jax.experimental import pallas as pl
from jax.experimental.pallas import tpu as pltpu

_B, _C, _H, _W = 4, 384, 224, 224
_S = 16
_T = 56
_G = 32
_CG = _C // _G


def _shift_kernel(x_ref, o_ref):
    g = pl.program_id(1)
    s = jnp.where(g >= _S, g - _S, 0)

    @pl.when(s == 0)
    def _():
        o_ref[...] = x_ref[...]

    for sv in range(1, _S):
        @pl.when(s == sv)
        def _(sv=sv):
            a, r = sv // 4, sv % 4
            for j1 in range(4):
                q1 = (r + j1) % 4
                k = (a + (r + j1) // 4) % 4
                lo, ql = j1 * _T, q1 * _T
                if k == 0:
                    o_ref[0, 0, :, :, lo:lo + _T] = x_ref[0, 0, :, :, ql:ql + _T]
                else:
                    o_ref[0, 0, :, : _H - _T * k, lo:lo + _T] = (
                        x_ref[0, 0, :, _T * k:, ql:ql + _T])
                    o_ref[0, 0, :, _H - _T * k:, lo:lo + _T] = (
                        x_ref[0, 0, :, : _T * k, ql:ql + _T])


def kernel(x):
    xr = x.reshape(_B, _G, _CG, _H, _W)
    spec = pl.BlockSpec(
        (1, 1, _CG, _H, _W), lambda b, g: (b, g, 0, 0, 0)
    )
    out = pl.pallas_call(
        _shift_kernel,
        grid=(_B, _G),
        in_specs=[spec],
        out_specs=spec,
        out_shape=jax.ShapeDtypeStruct((_B, _G, _CG, _H, _W), x.dtype),
        compiler_params=pltpu.CompilerParams(
            dimension_semantics=("arbitrary", "arbitrary"),
        ),
    )(xr)
    return out.reshape(_B, _C, _H, _W)
